# baseline (speedup 1.0000x reference)
import functools

import jax
import jax.numpy as jnp
from jax import lax
from jax.experimental import pallas as pl
from jax.experimental.pallas import tpu as pltpu
from jax.experimental.pallas import tpu_sc as plsc

B = 4096
L = 50
LP = 56
CHAR_D = 112
DIST_D = 16
OUT_D = CHAR_D + DIST_D
NC = 2
NS = 16
NW = NC * NS
BATCH_PER_W = B // NW
NBB = 2
NCHUNK = BATCH_PER_W // NBB
NBUF = 4
PFD = 2
KITER = NCHUNK // NBUF


def _concat_embed_sc(x_hbm, d_hbm, char_hbm, dist_hbm, out_hbm,
                     xi_v, dvi_v, dvals_v, *bufs):
    obuf = bufs[0:NBUF]
    cg = bufs[NBUF:2 * NBUF]
    cs = bufs[2 * NBUF:3 * NBUF]

    wid = lax.axis_index("s") * NC + lax.axis_index("c")
    wbase = wid * BATCH_PER_W
    pltpu.sync_copy(x_hbm.at[pl.ds(wbase * LP, BATCH_PER_W * LP)], xi_v)
    pltpu.sync_copy(d_hbm.at[pl.ds(wbase, BATCH_PER_W)], dvi_v)
    pltpu.async_copy(dist_hbm.at[dvi_v], dvals_v, cg[0]).wait()

    def issue_gathers(g, b):
        for bl in range(NBB):
            idx = xi_v.at[pl.ds((g * NBB + bl) * LP, LP)]
            pltpu.async_copy(char_hbm.at[idx], obuf[b].at[bl], cg[b])

    def wait_gathers(b):
        for bl in range(NBB):
            pltpu.make_async_copy(char_hbm.at[pl.ds(0, LP)],
                                  obuf[b].at[bl], cg[b]).wait()

    def issue_store(g, b):
        pltpu.async_copy(obuf[b].at[:, pl.ds(0, L), :],
                         out_hbm.at[pl.ds(wbase + g * NBB, NBB)], cs[b])

    def wait_store(b):
        pltpu.make_async_copy(obuf[b].at[:, pl.ds(0, L), :],
                              out_hbm.at[pl.ds(wbase, NBB)], cs[b]).wait()

    def fill_dist(g, b):
        ob = obuf[b]
        for bl in range(NBB):
            v = dvals_v[g * NBB + bl, pl.ds(0, DIST_D)]

            def fr(i, carry):
                for j in range(5):
                    ob[bl, i * 5 + j, pl.ds(CHAR_D, DIST_D)] = v
                return carry

            lax.fori_loop(0, L // 5, fr, 0)

    for b in range(PFD):
        issue_gathers(b, b)

    def body(k, carry):
        for b in range(NBUF):
            g = k * NBUF + b
            wait_gathers(b)
            fill_dist(g, b)
            issue_store(g, b)
            b2 = (b + PFD) % NBUF
            g2 = g + PFD
            if b + PFD < NBUF:
                @pl.when(k >= 1)
                def _():
                    wait_store(b2)
                    issue_gathers(g2, b2)

                @pl.when(k == 0)
                def _():
                    issue_gathers(g2, b2)
            else:
                @pl.when(k < KITER - 1)
                def _():
                    wait_store(b2)
                    issue_gathers(g2, b2)
        return carry

    lax.fori_loop(0, KITER, body, 0)

    for b in range(NBUF):
        wait_store(b)


@jax.jit
def _run(xpf, d, char128, dist128):
    mesh = plsc.VectorSubcoreMesh(core_axis_name="c", subcore_axis_name="s")
    scratch = [
        pltpu.VMEM((BATCH_PER_W * LP,), jnp.int32),
        pltpu.VMEM((BATCH_PER_W,), jnp.int32),
        pltpu.VMEM((BATCH_PER_W, OUT_D), jnp.float32),
    ]
    scratch += [pltpu.VMEM((NBB, LP, OUT_D), jnp.float32) for _ in range(NBUF)]
    scratch += [pltpu.SemaphoreType.DMA for _ in range(2 * NBUF)]
    f = functools.partial(
        pl.kernel,
        mesh=mesh,
        out_type=jax.ShapeDtypeStruct((B, L, OUT_D), jnp.float32),
        scratch_types=scratch,
    )(_concat_embed_sc)
    return f(xpf, d, char128, dist128)


def kernel(x, d, char_table, dist_table):
    xpf = jnp.pad(x, ((0, 0), (0, LP - L))).reshape(B * LP)
    char128 = jnp.pad(char_table, ((0, 0), (0, DIST_D)))
    dist128 = jnp.pad(dist_table, ((0, 0), (0, CHAR_D)))
    return _run(xpf, d, char128, dist128)


# speedup vs baseline: 2.5498x; 2.5498x over previous
import functools

import jax
import jax.numpy as jnp
from jax import lax
from jax.experimental import pallas as pl
from jax.experimental.pallas import tpu as pltpu
from jax.experimental.pallas import tpu_sc as plsc

B = 4096
L = 50
CHAR_D = 112
DIST_D = 16
OUT_D = CHAR_D + DIST_D
N_ROWS = B * L
NC = 2
NS = 16
NW = NC * NS
ROWS_PER_W = N_ROWS // NW
BATCH_PER_W = B // NW
G = 128
NCHUNK = ROWS_PER_W // G
NBUF = 5
PFD = 3
KITER = NCHUNK // NBUF


def _concat_embed_sc(x_hbm, d_hbm, char_hbm, dist_hbm, out_hbm,
                     xi_v, dvi_v, dvals_v, *bufs):
    orow = bufs[0:NBUF]
    cg = bufs[NBUF:2 * NBUF]
    cs = bufs[2 * NBUF:3 * NBUF]

    wid = lax.axis_index("s") * NC + lax.axis_index("c")
    base = wid * ROWS_PER_W
    pltpu.sync_copy(x_hbm.at[pl.ds(base, ROWS_PER_W)], xi_v)
    pltpu.sync_copy(d_hbm.at[pl.ds(wid * BATCH_PER_W, BATCH_PER_W)], dvi_v)
    pltpu.async_copy(dist_hbm.at[dvi_v], dvals_v, cg[0]).wait()


    def issue_gather(g, b):
        pltpu.async_copy(char_hbm.at[xi_v.at[pl.ds(g * G, G)]], orow[b], cg[b])

    def wait_gather(b):
        pltpu.make_async_copy(char_hbm.at[pl.ds(0, G)], orow[b], cg[b]).wait()

    def issue_store(g, b):
        pltpu.async_copy(orow[b], out_hbm.at[pl.ds(base + g * G, G)], cs[b])

    def wait_store(b):
        pltpu.make_async_copy(orow[b], out_hbm.at[pl.ds(base, G)], cs[b]).wait()

    def fill_dist(g, b):
        ob = orow[b]

        def fb(i, carry):
            for j in range(4):
                r = i * 4 + j
                lb = (g * G + r) // L
                ob[r, pl.ds(CHAR_D, DIST_D)] = dvals_v[lb, pl.ds(0, DIST_D)]
            return carry

        lax.fori_loop(0, G // 4, fb, 0)

    for b in range(PFD):
        issue_gather(b, b)

    def body(k, carry):
        for b in range(NBUF):
            g = k * NBUF + b
            wait_gather(b)
            fill_dist(g, b)
            issue_store(g, b)
            b3 = (b + PFD) % NBUF
            g3 = g + PFD
            if b + PFD < NBUF:
                @pl.when(k >= 1)
                def _():
                    wait_store(b3)
                    issue_gather(g3, b3)

                @pl.when(k == 0)
                def _():
                    issue_gather(g3, b3)
            else:
                @pl.when(k < KITER - 1)
                def _():
                    wait_store(b3)
                    issue_gather(g3, b3)
        return carry

    lax.fori_loop(0, KITER, body, 0)

    for b in range(NBUF):
        wait_store(b)


@jax.jit
def _run(xf, d, char_table, dist_table):
    mesh = plsc.VectorSubcoreMesh(core_axis_name="c", subcore_axis_name="s")
    scratch = [
        pltpu.VMEM((ROWS_PER_W,), jnp.int32),
        pltpu.VMEM((BATCH_PER_W,), jnp.int32),
        pltpu.VMEM((BATCH_PER_W, OUT_D), jnp.float32),
    ]
    scratch += [pltpu.VMEM((G, OUT_D), jnp.float32) for _ in range(NBUF)]
    scratch += [pltpu.SemaphoreType.DMA for _ in range(2 * NBUF)]
    f = functools.partial(
        pl.kernel,
        mesh=mesh,
        out_type=jax.ShapeDtypeStruct((N_ROWS, OUT_D), jnp.float32),
        scratch_types=scratch,
    )(_concat_embed_sc)
    return f(xf, d, char_table, dist_table)


def kernel(x, d, char_table, dist_table):
    xf = x.reshape(N_ROWS)
    c = jax.lax.optimization_barrier(jnp.float32(1.0))
    char128 = jnp.pad(char_table, ((0, 0), (0, DIST_D))) * c
    dist128 = jnp.pad(dist_table, ((0, 0), (0, CHAR_D)))
    out = _run(xf, d, char128, dist128)
    return out.reshape(B, L, OUT_D) * c


# speedup vs baseline: 4.6908x; 1.8397x over previous
import functools

import jax
import jax.numpy as jnp
from jax import lax
from jax.experimental import pallas as pl
from jax.experimental.pallas import tpu as pltpu
from jax.experimental.pallas import tpu_sc as plsc

B = 4096
L = 50
CHAR_D = 112
DIST_D = 16
OUT_D = CHAR_D + DIST_D
N_ROWS = B * L
NDIST = 201
NC = 2
NS = 16
NW = NC * NS
ROWS_PER_W = N_ROWS // NW
BATCH_PER_W = B // NW
G = 128
NCHUNK = L
NBUF = 5
PFD = 3
KITER = NCHUNK // NBUF


def _concat_embed_sc(x_hbm, d_hbm, char_hbm, dist_hbm, out_hbm,
                     xi_v, dvi_v, dexp_v, *bufs):
    orow = bufs[0:NBUF]
    cg = bufs[NBUF:2 * NBUF]
    cs = bufs[2 * NBUF:3 * NBUF]

    wid = lax.axis_index("s") * NC + lax.axis_index("c")
    base = wid * ROWS_PER_W
    bblk = wid * BATCH_PER_W
    pltpu.sync_copy(x_hbm.at[pl.ds(base, ROWS_PER_W)], xi_v)
    pltpu.sync_copy(d_hbm.at[pl.ds(bblk, BATCH_PER_W)], dvi_v)
    pltpu.async_copy(dist_hbm.at[dvi_v], dexp_v, cg[0]).wait()

    def issue_gather(g, b):
        pltpu.async_copy(char_hbm.at[xi_v.at[pl.ds(g * G, G)]], orow[b], cg[b])

    def wait_gather(b):
        pltpu.make_async_copy(char_hbm.at[pl.ds(0, G)], orow[b], cg[b]).wait()

    def issue_store(g, b):
        pltpu.async_copy(orow[b], out_hbm.at[g, pl.ds(bblk, G)], cs[b])

    def wait_store(b):
        pltpu.make_async_copy(orow[b], out_hbm.at[0, pl.ds(bblk, G)], cs[b]).wait()

    def fill_dist(b):
        ob = orow[b]

        def fb(i, carry):
            for j in range(4):
                r = i * 4 + j
                ob[r, pl.ds(CHAR_D, DIST_D)] = dexp_v[r, pl.ds(0, DIST_D)]
            return carry

        lax.fori_loop(0, G // 4, fb, 0)

    for b in range(PFD):
        issue_gather(b, b)

    def body(k, carry):
        for b in range(NBUF):
            g = k * NBUF + b
            wait_gather(b)
            fill_dist(b)
            issue_store(g, b)
            b3 = (b + PFD) % NBUF
            g3 = g + PFD
            if b + PFD < NBUF:
                @pl.when(k >= 1)
                def _():
                    wait_store(b3)
                    issue_gather(g3, b3)

                @pl.when(k == 0)
                def _():
                    issue_gather(g3, b3)
            else:
                @pl.when(k < KITER - 1)
                def _():
                    wait_store(b3)
                    issue_gather(g3, b3)
        return carry

    lax.fori_loop(0, KITER, body, 0)

    for b in range(NBUF):
        wait_store(b)


@jax.jit
def _run(xarr, d, char128, dist128):
    mesh = plsc.VectorSubcoreMesh(core_axis_name="c", subcore_axis_name="s")
    scratch = [
        pltpu.VMEM((ROWS_PER_W,), jnp.int32),
        pltpu.VMEM((BATCH_PER_W,), jnp.int32),
        pltpu.VMEM((BATCH_PER_W, OUT_D), jnp.float32),
    ]
    scratch += [pltpu.VMEM((G, OUT_D), jnp.float32) for _ in range(NBUF)]
    scratch += [pltpu.SemaphoreType.DMA for _ in range(2 * NBUF)]
    f = functools.partial(
        pl.kernel,
        mesh=mesh,
        out_type=jax.ShapeDtypeStruct((L, B, OUT_D), jnp.float32),
        scratch_types=scratch,
    )(_concat_embed_sc)
    return f(xarr, d, char128, dist128)


def kernel(x, d, char_table, dist_table):
    xarr = x.T.reshape(L, NW, BATCH_PER_W).swapaxes(0, 1).reshape(N_ROWS)
    char128 = jnp.pad(char_table, ((0, 0), (0, DIST_D)))
    dist128 = jnp.pad(dist_table, ((0, 0), (0, CHAR_D)))
    out_t = _run(xarr, d, char128, dist128)
    return jnp.swapaxes(out_t, 0, 1)


# speedup vs baseline: 8.4998x; 1.8120x over previous
import functools

import jax
import jax.numpy as jnp
from jax import lax
from jax.experimental import pallas as pl
from jax.experimental.pallas import tpu as pltpu
from jax.experimental.pallas import tpu_sc as plsc

B = 4096
L = 50
TRC = 1024
NTBLK = 98
NTAB = NTBLK * TRC
CHAR_D = 112
DIST_D = 16
OUT_D = CHAR_D + DIST_D
N_ROWS = B * L
NDIST = 201
NC = 2
NS = 16
NW = NC * NS
ROWS_PER_W = N_ROWS // NW
BATCH_PER_W = B // NW
G = 128
NCHUNK = L
NBUF = 5
PFD = 3
KITER = NCHUNK // NBUF


def _concat_embed_sc(x_hbm, d_hbm, char_hbm, dist_hbm, out_hbm,
                     xi_v, dvi_v, dexp_v, *bufs):
    orow = bufs[0:NBUF]
    cg = bufs[NBUF:2 * NBUF]
    cs = bufs[2 * NBUF:3 * NBUF]

    wid = lax.axis_index("s") * NC + lax.axis_index("c")
    base = wid * ROWS_PER_W
    bblk = wid * BATCH_PER_W
    pltpu.sync_copy(x_hbm.at[pl.ds(base, ROWS_PER_W)], xi_v)
    pltpu.sync_copy(d_hbm.at[pl.ds(bblk, BATCH_PER_W)], dvi_v)
    pltpu.async_copy(dist_hbm.at[dvi_v], dexp_v, cg[0]).wait()

    def issue_gather(g, b):
        pltpu.async_copy(char_hbm.at[xi_v.at[pl.ds(g * G, G)]], orow[b], cg[b])

    def wait_gather(b):
        pltpu.make_async_copy(char_hbm.at[pl.ds(0, G)], orow[b], cg[b]).wait()

    def issue_store(g, b):
        pltpu.async_copy(orow[b], out_hbm.at[g, pl.ds(bblk, G)], cs[b])

    def wait_store(b):
        pltpu.make_async_copy(orow[b], out_hbm.at[0, pl.ds(bblk, G)], cs[b]).wait()

    def fill_dist(b):
        ob = orow[b]

        def fb(i, carry):
            for j in range(4):
                r = i * 4 + j
                ob[r, pl.ds(CHAR_D, DIST_D)] = dexp_v[r, pl.ds(0, DIST_D)]
            return carry

        lax.fori_loop(0, G // 4, fb, 0)

    for b in range(PFD):
        issue_gather(b, b)

    def body(k, carry):
        for b in range(NBUF):
            g = k * NBUF + b
            wait_gather(b)
            fill_dist(b)
            issue_store(g, b)
            b3 = (b + PFD) % NBUF
            g3 = g + PFD
            if b + PFD < NBUF:
                @pl.when(k >= 1)
                def _():
                    wait_store(b3)
                    issue_gather(g3, b3)

                @pl.when(k == 0)
                def _():
                    issue_gather(g3, b3)
            else:
                @pl.when(k < KITER - 1)
                def _():
                    wait_store(b3)
                    issue_gather(g3, b3)
        return carry

    lax.fori_loop(0, KITER, body, 0)

    for b in range(NBUF):
        wait_store(b)


def _tr_body(in_ref, out_ref):
    blk = in_ref[...]
    out_ref[...] = jnp.pad(jnp.swapaxes(blk, 0, 1), ((0, 0), (0, DIST_D)))


@jax.jit
def _transpose_pad(charT):
    return pl.pallas_call(
        _tr_body,
        grid=(NTBLK,),
        in_specs=[pl.BlockSpec((CHAR_D, TRC), lambda i: (0, i))],
        out_specs=pl.BlockSpec((TRC, OUT_D), lambda i: (i, 0)),
        out_shape=jax.ShapeDtypeStruct((NTAB, OUT_D), jnp.float32),
    )(charT)


@jax.jit
def _run(xarr, d, char128, dist128):
    mesh = plsc.VectorSubcoreMesh(core_axis_name="c", subcore_axis_name="s")
    scratch = [
        pltpu.VMEM((ROWS_PER_W,), jnp.int32),
        pltpu.VMEM((BATCH_PER_W,), jnp.int32),
        pltpu.VMEM((BATCH_PER_W, OUT_D), jnp.float32),
    ]
    scratch += [pltpu.VMEM((G, OUT_D), jnp.float32) for _ in range(NBUF)]
    scratch += [pltpu.SemaphoreType.DMA for _ in range(2 * NBUF)]
    f = functools.partial(
        pl.kernel,
        mesh=mesh,
        out_type=jax.ShapeDtypeStruct((L, B, OUT_D), jnp.float32),
        scratch_types=scratch,
    )(_concat_embed_sc)
    return f(xarr, d, char128, dist128)


def kernel(x, d, char_table, dist_table):
    xarr = x.T.reshape(L, NW, BATCH_PER_W).swapaxes(0, 1).reshape(N_ROWS)
    char128 = _transpose_pad(char_table.T)
    dist128 = jnp.pad(dist_table, ((0, 0), (0, CHAR_D)))
    out_t = _run(xarr, d, char128, dist128)
    return jnp.swapaxes(out_t, 0, 1)


# speedup vs baseline: 9.8076x; 1.1539x over previous
import functools

import jax
import jax.numpy as jnp
from jax import lax
from jax.experimental import pallas as pl
from jax.experimental.pallas import tpu as pltpu
from jax.experimental.pallas import tpu_sc as plsc

B = 4096
L = 50
TRC = 2048
NTBLK = 49
NTAB = NTBLK * TRC
CHAR_D = 112
DIST_D = 16
OUT_D = CHAR_D + DIST_D
N_ROWS = B * L
NDIST = 201
NC = 2
NS = 16
NW = NC * NS
ROWS_PER_W = N_ROWS // NW
BATCH_PER_W = B // NW
G = 128
NCHUNK = L
NBUF = 5
PFD = 3
KITER = NCHUNK // NBUF


def _concat_embed_sc(x_hbm, d_hbm, char_hbm, dist_hbm, out_hbm,
                     xi_v, dvi_v, dexp_v, *bufs):
    orow = bufs[0:NBUF]
    cg = bufs[NBUF:2 * NBUF]
    cs = bufs[2 * NBUF:3 * NBUF]

    wid = lax.axis_index("s") * NC + lax.axis_index("c")
    base = wid * ROWS_PER_W
    bblk = wid * BATCH_PER_W
    pltpu.sync_copy(x_hbm.at[pl.ds(base, ROWS_PER_W)], xi_v)
    pltpu.sync_copy(d_hbm.at[pl.ds(bblk, BATCH_PER_W)], dvi_v)
    pltpu.async_copy(dist_hbm.at[dvi_v], dexp_v, cg[0]).wait()

    def issue_gather(g, b):
        pltpu.async_copy(char_hbm.at[xi_v.at[pl.ds(g * G, G)]], orow[b], cg[b])

    def wait_gather(b):
        pltpu.make_async_copy(char_hbm.at[pl.ds(0, G)], orow[b], cg[b]).wait()

    def issue_store(g, b):
        pltpu.async_copy(orow[b], out_hbm.at[g, pl.ds(bblk, G)], cs[b])

    def wait_store(b):
        pltpu.make_async_copy(orow[b], out_hbm.at[0, pl.ds(bblk, G)], cs[b]).wait()

    def fill_dist(b):
        ob = orow[b]

        def fb(i, carry):
            for j in range(4):
                r = i * 4 + j
                ob[r, pl.ds(CHAR_D, DIST_D)] = dexp_v[r, pl.ds(0, DIST_D)]
            return carry

        lax.fori_loop(0, G // 4, fb, 0)

    for b in range(PFD):
        issue_gather(b, b)

    def body(k, carry):
        for b in range(NBUF):
            g = k * NBUF + b
            wait_gather(b)
            fill_dist(b)
            issue_store(g, b)
            b3 = (b + PFD) % NBUF
            g3 = g + PFD
            if b + PFD < NBUF:
                @pl.when(k >= 1)
                def _():
                    wait_store(b3)
                    issue_gather(g3, b3)

                @pl.when(k == 0)
                def _():
                    issue_gather(g3, b3)
            else:
                @pl.when(k < KITER - 1)
                def _():
                    wait_store(b3)
                    issue_gather(g3, b3)
        return carry

    lax.fori_loop(0, KITER, body, 0)

    for b in range(NBUF):
        wait_store(b)


def _tr_body(in_ref, out_ref):
    blk = in_ref[...]
    out_ref[...] = jnp.pad(jnp.swapaxes(blk, 0, 1), ((0, 0), (0, DIST_D)))


@jax.jit
def _transpose_pad(charT):
    return pl.pallas_call(
        _tr_body,
        grid=(NTBLK,),
        in_specs=[pl.BlockSpec((CHAR_D, TRC), lambda i: (0, i))],
        out_specs=pl.BlockSpec((TRC, OUT_D), lambda i: (i, 0)),
        out_shape=jax.ShapeDtypeStruct((NTAB, OUT_D), jnp.float32),
    )(charT)


@jax.jit
def _run(xarr, d, char128, dist128):
    mesh = plsc.VectorSubcoreMesh(core_axis_name="c", subcore_axis_name="s")
    scratch = [
        pltpu.VMEM((ROWS_PER_W,), jnp.int32),
        pltpu.VMEM((BATCH_PER_W,), jnp.int32),
        pltpu.VMEM((BATCH_PER_W, OUT_D), jnp.float32),
    ]
    scratch += [pltpu.VMEM((G, OUT_D), jnp.float32) for _ in range(NBUF)]
    scratch += [pltpu.SemaphoreType.DMA for _ in range(2 * NBUF)]
    f = functools.partial(
        pl.kernel,
        mesh=mesh,
        out_type=jax.ShapeDtypeStruct((L, B, OUT_D), jnp.float32),
        scratch_types=scratch,
    )(_concat_embed_sc)
    return f(xarr, d, char128, dist128)


def kernel(x, d, char_table, dist_table):
    xarr = x.T.reshape(L, NW, BATCH_PER_W).swapaxes(0, 1).reshape(N_ROWS)
    char128 = _transpose_pad(char_table.T)
    dist128 = jnp.pad(dist_table, ((0, 0), (0, CHAR_D)))
    out_t = _run(xarr, d, char128, dist128)
    return jnp.swapaxes(out_t, 0, 1)


# speedup vs baseline: 10.8577x; 1.1071x over previous
import functools

import jax
import jax.numpy as jnp
from jax import lax
from jax.experimental import pallas as pl
from jax.experimental.pallas import tpu as pltpu
from jax.experimental.pallas import tpu_sc as plsc

B = 4096
L = 50
TRC = 4096
NTBLK = 25
NTAB = NTBLK * TRC
CHAR_D = 112
DIST_D = 16
OUT_D = CHAR_D + DIST_D
N_ROWS = B * L
NDIST = 201
NC = 2
NS = 16
NW = NC * NS
ROWS_PER_W = N_ROWS // NW
BATCH_PER_W = B // NW
G = 128
NCHUNK = L
NBUF = 5
PFD = 3
KITER = NCHUNK // NBUF


def _concat_embed_sc(x_hbm, d_hbm, char_hbm, dist_hbm, out_hbm,
                     xi_v, dvi_v, dexp_v, *bufs):
    orow = bufs[0:NBUF]
    cg = bufs[NBUF:2 * NBUF]
    cs = bufs[2 * NBUF:3 * NBUF]

    wid = lax.axis_index("s") * NC + lax.axis_index("c")
    base = wid * ROWS_PER_W
    bblk = wid * BATCH_PER_W
    pltpu.sync_copy(x_hbm.at[pl.ds(base, ROWS_PER_W)], xi_v)
    pltpu.sync_copy(d_hbm.at[pl.ds(bblk, BATCH_PER_W)], dvi_v)
    pltpu.async_copy(dist_hbm.at[dvi_v], dexp_v, cg[0]).wait()

    def issue_gather(g, b):
        pltpu.async_copy(char_hbm.at[xi_v.at[pl.ds(g * G, G)]], orow[b], cg[b])

    def wait_gather(b):
        pltpu.make_async_copy(char_hbm.at[pl.ds(0, G)], orow[b], cg[b]).wait()

    def issue_store(g, b):
        pltpu.async_copy(orow[b], out_hbm.at[g, pl.ds(bblk, G)], cs[b])

    def wait_store(b):
        pltpu.make_async_copy(orow[b], out_hbm.at[0, pl.ds(bblk, G)], cs[b]).wait()

    def fill_dist(b):
        ob = orow[b]

        def fb(i, carry):
            for j in range(4):
                r = i * 4 + j
                ob[r, pl.ds(CHAR_D, DIST_D)] = dexp_v[r, pl.ds(0, DIST_D)]
            return carry

        lax.fori_loop(0, G // 4, fb, 0)

    for b in range(PFD):
        issue_gather(b, b)

    def body(k, carry):
        for b in range(NBUF):
            g = k * NBUF + b
            wait_gather(b)
            fill_dist(b)
            issue_store(g, b)
            b3 = (b + PFD) % NBUF
            g3 = g + PFD
            if b + PFD < NBUF:
                @pl.when(k >= 1)
                def _():
                    wait_store(b3)
                    issue_gather(g3, b3)

                @pl.when(k == 0)
                def _():
                    issue_gather(g3, b3)
            else:
                @pl.when(k < KITER - 1)
                def _():
                    wait_store(b3)
                    issue_gather(g3, b3)
        return carry

    lax.fori_loop(0, KITER, body, 0)

    for b in range(NBUF):
        wait_store(b)


def _tr_body(in_ref, out_ref):
    blk = in_ref[...]
    out_ref[...] = jnp.pad(jnp.swapaxes(blk, 0, 1), ((0, 0), (0, DIST_D)))


@jax.jit
def _transpose_pad(charT):
    return pl.pallas_call(
        _tr_body,
        grid=(NTBLK,),
        in_specs=[pl.BlockSpec((CHAR_D, TRC), lambda i: (0, i))],
        out_specs=pl.BlockSpec((TRC, OUT_D), lambda i: (i, 0)),
        out_shape=jax.ShapeDtypeStruct((NTAB, OUT_D), jnp.float32),
    )(charT)


@jax.jit
def _run(xarr, d, char128, dist128):
    mesh = plsc.VectorSubcoreMesh(core_axis_name="c", subcore_axis_name="s")
    scratch = [
        pltpu.VMEM((ROWS_PER_W,), jnp.int32),
        pltpu.VMEM((BATCH_PER_W,), jnp.int32),
        pltpu.VMEM((BATCH_PER_W, OUT_D), jnp.float32),
    ]
    scratch += [pltpu.VMEM((G, OUT_D), jnp.float32) for _ in range(NBUF)]
    scratch += [pltpu.SemaphoreType.DMA for _ in range(2 * NBUF)]
    f = functools.partial(
        pl.kernel,
        mesh=mesh,
        out_type=jax.ShapeDtypeStruct((L, B, OUT_D), jnp.float32),
        scratch_types=scratch,
    )(_concat_embed_sc)
    return f(xarr, d, char128, dist128)


def kernel(x, d, char_table, dist_table):
    xarr = x.T.reshape(L, NW, BATCH_PER_W).swapaxes(0, 1).reshape(N_ROWS)
    char128 = _transpose_pad(char_table.T)
    dist128 = jnp.pad(dist_table, ((0, 0), (0, CHAR_D)))
    out_t = _run(xarr, d, char128, dist128)
    return jnp.swapaxes(out_t, 0, 1)


# speedup vs baseline: 11.2568x; 1.0368x over previous
import functools

import jax
import jax.numpy as jnp
from jax import lax
from jax.experimental import pallas as pl
from jax.experimental.pallas import tpu as pltpu
from jax.experimental.pallas import tpu_sc as plsc

B = 4096
L = 50
TRC = 8192
NTBLK = 13
NTAB = NTBLK * TRC
CHAR_D = 112
DIST_D = 16
OUT_D = CHAR_D + DIST_D
N_ROWS = B * L
NDIST = 201
NC = 2
NS = 16
NW = NC * NS
ROWS_PER_W = N_ROWS // NW
BATCH_PER_W = B // NW
G = 128
NCHUNK = L
NBUF = 5
PFD = 3
KITER = NCHUNK // NBUF


def _concat_embed_sc(x_hbm, d_hbm, char_hbm, dist_hbm, out_hbm,
                     xi_v, dvi_v, dexp_v, *bufs):
    orow = bufs[0:NBUF]
    cg = bufs[NBUF:2 * NBUF]
    cs = bufs[2 * NBUF:3 * NBUF]

    wid = lax.axis_index("s") * NC + lax.axis_index("c")
    base = wid * ROWS_PER_W
    bblk = wid * BATCH_PER_W
    pltpu.sync_copy(x_hbm.at[pl.ds(base, ROWS_PER_W)], xi_v)
    pltpu.sync_copy(d_hbm.at[pl.ds(bblk, BATCH_PER_W)], dvi_v)
    pltpu.async_copy(dist_hbm.at[dvi_v], dexp_v, cg[0]).wait()

    def issue_gather(g, b):
        pltpu.async_copy(char_hbm.at[xi_v.at[pl.ds(g * G, G)]], orow[b], cg[b])

    def wait_gather(b):
        pltpu.make_async_copy(char_hbm.at[pl.ds(0, G)], orow[b], cg[b]).wait()

    def issue_store(g, b):
        pltpu.async_copy(orow[b], out_hbm.at[g, pl.ds(bblk, G)], cs[b])

    def wait_store(b):
        pltpu.make_async_copy(orow[b], out_hbm.at[0, pl.ds(bblk, G)], cs[b]).wait()

    def fill_dist(b):
        ob = orow[b]

        def fb(i, carry):
            for j in range(4):
                r = i * 4 + j
                ob[r, pl.ds(CHAR_D, DIST_D)] = dexp_v[r, pl.ds(0, DIST_D)]
            return carry

        lax.fori_loop(0, G // 4, fb, 0)

    for b in range(PFD):
        issue_gather(b, b)

    def body(k, carry):
        for b in range(NBUF):
            g = k * NBUF + b
            wait_gather(b)
            fill_dist(b)
            issue_store(g, b)
            b3 = (b + PFD) % NBUF
            g3 = g + PFD
            if b + PFD < NBUF:
                @pl.when(k >= 1)
                def _():
                    wait_store(b3)
                    issue_gather(g3, b3)

                @pl.when(k == 0)
                def _():
                    issue_gather(g3, b3)
            else:
                @pl.when(k < KITER - 1)
                def _():
                    wait_store(b3)
                    issue_gather(g3, b3)
        return carry

    lax.fori_loop(0, KITER, body, 0)

    for b in range(NBUF):
        wait_store(b)


def _tr_body(in_ref, out_ref):
    blk = in_ref[...]
    out_ref[...] = jnp.pad(jnp.swapaxes(blk, 0, 1), ((0, 0), (0, DIST_D)))


@jax.jit
def _transpose_pad(charT):
    return pl.pallas_call(
        _tr_body,
        grid=(NTBLK,),
        in_specs=[pl.BlockSpec((CHAR_D, TRC), lambda i: (0, i))],
        out_specs=pl.BlockSpec((TRC, OUT_D), lambda i: (i, 0)),
        out_shape=jax.ShapeDtypeStruct((NTAB, OUT_D), jnp.float32),
    )(charT)


@jax.jit
def _run(xarr, d, char128, dist128):
    mesh = plsc.VectorSubcoreMesh(core_axis_name="c", subcore_axis_name="s")
    scratch = [
        pltpu.VMEM((ROWS_PER_W,), jnp.int32),
        pltpu.VMEM((BATCH_PER_W,), jnp.int32),
        pltpu.VMEM((BATCH_PER_W, OUT_D), jnp.float32),
    ]
    scratch += [pltpu.VMEM((G, OUT_D), jnp.float32) for _ in range(NBUF)]
    scratch += [pltpu.SemaphoreType.DMA for _ in range(2 * NBUF)]
    f = functools.partial(
        pl.kernel,
        mesh=mesh,
        out_type=jax.ShapeDtypeStruct((L, B, OUT_D), jnp.float32),
        scratch_types=scratch,
    )(_concat_embed_sc)
    return f(xarr, d, char128, dist128)


def kernel(x, d, char_table, dist_table):
    xarr = x.T.reshape(L, NW, BATCH_PER_W).swapaxes(0, 1).reshape(N_ROWS)
    char128 = _transpose_pad(char_table.T)
    dist128 = jnp.pad(dist_table, ((0, 0), (0, CHAR_D)))
    out_t = _run(xarr, d, char128, dist128)
    return jnp.swapaxes(out_t, 0, 1)
